# baseline (speedup 1.0000x reference)
import functools

import jax
import jax.numpy as jnp
from jax import lax
from jax.experimental import pallas as pl
from jax.experimental.pallas import tpu as pltpu
from jax.experimental.pallas import tpu_sc as plsc

N_NODES = 10000
N_EDGES = 160000
D_IN = 256
H = 128
NC = 2
NS = 16
L = 16
EDGES_PER_TILE = 10240
EPAD = NS * EDGES_PER_TILE
ICH = 256
NCH_I = EDGES_PER_TILE // ICH
HALF_N = 5056
TRASH = HALF_N
ACC_ROWS = 5120
GCH = 48
NCHG = 118
CAP = NCHG * GCH
CAPP = 5760
SLAB = 632
NPAD = NC * HALF_N
HISTN = 5120


def _sc_aggregate(features, idxpk, zrows, zhist, zidx):
    mesh = plsc.VectorSubcoreMesh(
        core_axis_name="c", subcore_axis_name="s", num_cores=NC, num_subcores=NS
    )

    @functools.partial(
        pl.kernel,
        out_type=[
            jax.ShapeDtypeStruct((2 * NPAD, H), jnp.float32),
            jax.ShapeDtypeStruct((NS * NC, HISTN), jnp.float32),
        ],
        mesh=mesh,
        compiler_params=pltpu.CompilerParams(needs_layout_passes=False),
        scratch_types=[
            pltpu.VMEM((CAPP,), jnp.int32),
            pltpu.VMEM((CAPP,), jnp.int32),
            pltpu.VMEM_SHARED((ACC_ROWS, H), jnp.float32),
            pltpu.VMEM_SHARED((ACC_ROWS, H), jnp.float32),
            pltpu.VMEM((2, ICH), jnp.int32),
            pltpu.VMEM((2, ICH), jnp.int32),
            pltpu.VMEM((GCH, H), jnp.float32),
            pltpu.VMEM((GCH, H), jnp.float32),
            pltpu.VMEM((GCH, H), jnp.float32),
            pltpu.VMEM((GCH, H), jnp.float32),
            pltpu.VMEM((2 * GCH,), jnp.int32),
            pltpu.VMEM((2 * GCH,), jnp.int32),
            pltpu.VMEM((HISTN,), jnp.float32),
            pltpu.SemaphoreType.DMA,
            pltpu.SemaphoreType.DMA,
            pltpu.SemaphoreType.DMA,
            pltpu.SemaphoreType.DMA,
            pltpu.SemaphoreType.DMA,
            pltpu.SemaphoreType.DMA,
        ],
    )
    def body(feat_hbm, idx_hbm, z_hbm, zh_hbm, zi_hbm, agg_hbm, deg_hbm,
             srcl, dstl, acc0, acc1, ibuf0, ibuf1, rowsA0, rowsA1, rowsB0,
             rowsB1, sidx0, sidx1, hist, isem0, isem1, gsemA0, gsemA1,
             gsemB0, gsemB1):
        c = lax.axis_index("c")
        s = lax.axis_index("s")
        node_base = c * HALF_N

        @pl.when(s < 8)
        def _():
            pltpu.sync_copy(z_hbm, acc0.at[pl.ds(s * SLAB, SLAB)])
        @pl.when(s >= 8)
        def _():
            pltpu.sync_copy(z_hbm, acc1.at[pl.ds((s - 8) * SLAB, SLAB)])
        pltpu.sync_copy(zh_hbm, hist)
        pltpu.sync_copy(zi_hbm, sidx0)
        pltpu.sync_copy(zi_hbm, sidx1)

        zi = jnp.zeros((L,), jnp.int32)
        ti = jnp.full((L,), TRASH, jnp.int32)

        def prefill(r, _):
            srcl[pl.ds(r * L, L)] = zi
            dstl[pl.ds(r * L, L)] = ti
            return 0

        lax.fori_loop(0, CAPP // L, prefill, 0)

        ibufs = (ibuf0, ibuf1)
        isems = (isem0, isem1)

        def iload(m, slot):
            return pltpu.async_copy(idx_hbm.at[s, m], ibufs[slot], isems[slot])

        def iwait(m, slot):
            pltpu.make_async_copy(idx_hbm.at[s, m], ibufs[slot],
                                  isems[slot]).wait()

        iload(0, 0)

        def pchunk(m2, off0):
            off = off0
            for b in range(2):
                m = m2 * 2 + b
                nb = 1 - b
                @pl.when(m + 1 < NCH_I)
                def _():
                    iload(m + 1, nb)
                iwait(m, b)
                for j in range(ICH // L):
                    srcv = ibufs[b][0, pl.ds(j * L, L)]
                    dstv = ibufs[b][1, pl.ds(j * L, L)]
                    loc = dstv - node_base
                    mk = (loc >= 0) & (loc < HALF_N)
                    pref = lax.cumsum(mk.astype(jnp.int32), axis=0)
                    pos = off + pref - 1
                    mk = mk & (pos < CAP)
                    plsc.store_scatter(srcl, [pos], srcv, mask=mk)
                    plsc.store_scatter(dstl, [pos], loc, mask=mk)
                    off = off + pref[L - 1]
            return off

        lax.fori_loop(0, NCH_I // 2, pchunk, 0)

        plsc.subcore_barrier()

        rowsA = (rowsA0, rowsA1)
        rowsB = (rowsB0, rowsB1)
        sidxs = (sidx0, sidx1)
        gsemsA = (gsemA0, gsemA1)
        gsemsB = (gsemB0, gsemB1)
        lanes = lax.iota(jnp.int32, L)

        def stage(r, slot):
            for j in range(GCH // L):
                plsc.store_scatter(sidxs[slot], [lanes + j * L],
                                   srcl[pl.ds(r * GCH + j * L, L)])
                plsc.store_scatter(sidxs[slot], [lanes + (GCH + j * L)],
                                   dstl[pl.ds(r * GCH + j * L, L)])

        def gathers(slot):
            pltpu.async_copy(
                feat_hbm.at[sidxs[slot].at[pl.ds(0, GCH)], pl.ds(0, H)],
                rowsA[slot], gsemsA[slot])
            pltpu.async_copy(
                feat_hbm.at[sidxs[slot].at[pl.ds(0, GCH)], pl.ds(H, H)],
                rowsB[slot], gsemsB[slot])

        def gwait(slot):
            pltpu.make_async_copy(
                feat_hbm.at[sidxs[slot].at[pl.ds(0, GCH)], pl.ds(0, H)],
                rowsA[slot], gsemsA[slot]).wait()
            pltpu.make_async_copy(
                feat_hbm.at[sidxs[slot].at[pl.ds(0, GCH)], pl.ds(H, H)],
                rowsB[slot], gsemsB[slot]).wait()

        stage(0, 0)
        gathers(0)

        def gchunk(r2, _):
            for b in range(2):
                r = r2 * 2 + b
                nb = 1 - b
                @pl.when(r + 1 < NCHG)
                def _():
                    stage(r + 1, nb)
                    gathers(nb)
                for j in range(GCH // L):
                    d = dstl[pl.ds(r * GCH + j * L, L)]
                    cnt, last = plsc.scan_count(d)
                    plsc.addupdate_scatter(hist, [d], cnt.astype(jnp.float32),
                                           mask=last)
                gwait(b)
                pltpu.sync_copy(rowsA[b], acc0.at[sidxs[b].at[pl.ds(GCH, GCH)]],
                                add=True)
                pltpu.sync_copy(rowsB[b], acc1.at[sidxs[b].at[pl.ds(GCH, GCH)]],
                                add=True)
            return 0

        lax.fori_loop(0, NCHG // 2, gchunk, 0)

        plsc.subcore_barrier()
        @pl.when(s < 8)
        def _():
            pltpu.sync_copy(
                acc0.at[pl.ds(s * SLAB, SLAB)],
                agg_hbm.at[pl.ds(c * HALF_N + s * SLAB, SLAB)],
            )
        @pl.when(s >= 8)
        def _():
            pltpu.sync_copy(
                acc1.at[pl.ds((s - 8) * SLAB, SLAB)],
                agg_hbm.at[pl.ds(NPAD + c * HALF_N + (s - 8) * SLAB, SLAB)],
            )
        pltpu.sync_copy(hist, deg_hbm.at[s * NC + c])

    return body(features, idxpk, zrows, zhist, zidx)


BR = SLAB


def _tc_body(aggA, aggB, degp, feat, w0, w1, b, out):
    deg = jnp.sum(degp[...], axis=1)[:, None]
    scale = 1.0 / jnp.maximum(deg, 1.0)
    h = jnp.dot(aggA[...] * scale, w0[...], preferred_element_type=jnp.float32)
    h = h + jnp.dot(aggB[...] * scale, w1[...], preferred_element_type=jnp.float32)
    out[...] = jnp.maximum(h + b[...] + feat[...], 0.0)


def _tc_finish(agg, deg_parts, features, W, b):
    grid = (NPAD // BR,)
    return pl.pallas_call(
        _tc_body,
        grid=grid,
        in_specs=[
            pl.BlockSpec((BR, H), lambda i: (i, 0)),
            pl.BlockSpec((BR, H), lambda i: (i + NPAD // BR, 0)),
            pl.BlockSpec((BR, NS), lambda i: (i, 0)),
            pl.BlockSpec((BR, D_IN), lambda i: (i, 0)),
            pl.BlockSpec((H, D_IN), lambda i: (0, 0)),
            pl.BlockSpec((H, D_IN), lambda i: (0, 0)),
            pl.BlockSpec((1, D_IN), lambda i: (0, 0)),
        ],
        out_specs=pl.BlockSpec((BR, D_IN), lambda i: (i, 0)),
        out_shape=jax.ShapeDtypeStruct((N_NODES, D_IN), jnp.float32),
    )(agg, agg, deg_parts, features, W[:H], W[H:], b.reshape(1, D_IN))


def kernel(features, edges, W, b):
    src = edges[0].astype(jnp.int32)
    dst = edges[1].astype(jnp.int32)
    pad = EPAD - N_EDGES
    srcp = jnp.concatenate([src, jnp.zeros((pad,), jnp.int32)])
    dstp = jnp.concatenate([dst, jnp.full((pad,), NPAD, jnp.int32)])
    idxpk = jnp.stack(
        [srcp.reshape(NS, NCH_I, ICH), dstp.reshape(NS, NCH_I, ICH)], axis=2
    )
    zrows = jnp.zeros((SLAB, H), jnp.float32)
    zhist = jnp.zeros((HISTN,), jnp.float32)
    zidx = jnp.zeros((2 * GCH,), jnp.int32)
    agg, deg_parts = _sc_aggregate(features, idxpk, zrows, zhist, zidx)
    deg_parts = deg_parts.reshape(NS, NC, HISTN)[:, :, :HALF_N]
    deg_parts = deg_parts.transpose(1, 2, 0).reshape(NPAD, NS)
    return _tc_finish(agg, deg_parts, features, W, b)


# speedup vs baseline: 2.6547x; 2.6547x over previous
import functools

import jax
import jax.numpy as jnp
from jax import lax
from jax.experimental import pallas as pl
from jax.experimental.pallas import tpu as pltpu
from jax.experimental.pallas import tpu_sc as plsc

N_NODES = 10000
N_EDGES = 160000
D_IN = 256
H = 128
NC = 2
NS = 16
L = 16
CHUNK = 80
DEPTH = 3
EDGES_PER_TILE = 10240
EPAD = NS * EDGES_PER_TILE
NCHUNK = EDGES_PER_TILE // CHUNK
ROWS_PER_TILE = 640
NPAD = NS * ROWS_PER_TILE


def _sc_aggregate(features, idxpk, zrows, zhist):
    mesh = plsc.VectorSubcoreMesh(
        core_axis_name="c", subcore_axis_name="s", num_cores=NC, num_subcores=NS
    )

    @functools.partial(
        pl.kernel,
        out_type=[
            jax.ShapeDtypeStruct((NC * NPAD, H), jnp.float32),
            jax.ShapeDtypeStruct((NS, NPAD), jnp.float32),
        ],
        mesh=mesh,
        compiler_params=pltpu.CompilerParams(needs_layout_passes=False),
        scratch_types=(
            [pltpu.VMEM((2, CHUNK), jnp.int32) for _ in range(DEPTH)]
            + [pltpu.VMEM((CHUNK, H), jnp.float32) for _ in range(DEPTH)]
            + [
                pltpu.VMEM((NPAD,), jnp.float32),
                pltpu.VMEM_SHARED((NPAD, H), jnp.float32),
            ]
            + [pltpu.SemaphoreType.DMA for _ in range(2 * DEPTH)]
        ),
    )
    def body(feat_hbm, idx_hbm, z_hbm, zh_hbm, agg_hbm, deg_hbm, *refs):
        ibufs = refs[0:DEPTH]
        rowss = refs[DEPTH:2 * DEPTH]
        hist = refs[2 * DEPTH]
        acc = refs[2 * DEPTH + 1]
        isems = refs[2 * DEPTH + 2:2 * DEPTH + 2 + DEPTH]
        gsems = refs[2 * DEPTH + 2 + DEPTH:]
        c = lax.axis_index("c")
        s = lax.axis_index("s")
        colb = pl.multiple_of(c * H, H)
        pltpu.sync_copy(z_hbm, acc.at[pl.ds(s * ROWS_PER_TILE, ROWS_PER_TILE)])
        pltpu.sync_copy(zh_hbm, hist)
        plsc.subcore_barrier()

        def idx_load(k, slot):
            return pltpu.async_copy(idx_hbm.at[s, k], ibufs[slot], isems[slot])

        def idx_wait(k, slot):
            pltpu.make_async_copy(idx_hbm.at[s, k], ibufs[slot],
                                  isems[slot]).wait()

        def gather(slot):
            return pltpu.async_copy(
                feat_hbm.at[ibufs[slot].at[0], pl.ds(colb, H)], rowss[slot],
                gsems[slot])

        def gather_wait(slot):
            pltpu.make_async_copy(
                feat_hbm.at[ibufs[slot].at[0], pl.ds(colb, H)], rowss[slot],
                gsems[slot]).wait()

        for i in range(DEPTH):
            idx_load(i, i)
        for i in range(DEPTH - 1):
            idx_wait(i, i)
            gather(i)

        def group(m, _):
            for b in range(DEPTH):
                k = m * DEPTH + b
                lead = (b + DEPTH - 1) % DEPTH
                @pl.when(k + DEPTH - 1 < NCHUNK)
                def _():
                    idx_wait(k + DEPTH - 1, lead)
                    gather(lead)
                for j in range(CHUNK // L):
                    d = ibufs[b][1, pl.ds(j * L, L)]
                    cnt, last = plsc.scan_count(d)
                    plsc.addupdate_scatter(hist, [d], cnt.astype(jnp.float32),
                                           mask=last)
                gather_wait(b)
                pltpu.sync_copy(rowss[b], acc.at[ibufs[b].at[1]], add=True)
                @pl.when(k + DEPTH < NCHUNK)
                def _():
                    idx_load(k + DEPTH, b)
            return 0

        lax.fori_loop(0, NCHUNK // DEPTH, group, 0)
        plsc.subcore_barrier()
        pltpu.sync_copy(
            acc.at[pl.ds(s * ROWS_PER_TILE, ROWS_PER_TILE)],
            agg_hbm.at[pl.ds(c * NPAD + s * ROWS_PER_TILE, ROWS_PER_TILE)],
        )
        @pl.when(c == 0)
        def _():
            pltpu.sync_copy(hist, deg_hbm.at[s])

    return body(features, idxpk, zrows, zhist)


BR = ROWS_PER_TILE


def _tc_body(aggA, aggB, degp, feat, w0, w1, b, out):
    deg = jnp.sum(degp[...], axis=0)[:, None]
    scale = 1.0 / jnp.maximum(deg, 1.0)
    h = jnp.dot(aggA[...] * scale, w0[...], preferred_element_type=jnp.float32)
    h = h + jnp.dot(aggB[...] * scale, w1[...], preferred_element_type=jnp.float32)
    out[...] = jnp.maximum(h + b[...] + feat[...], 0.0)


def _tc_finish(agg, deg_parts, features, W, b):
    grid = (NPAD // BR,)
    return pl.pallas_call(
        _tc_body,
        grid=grid,
        in_specs=[
            pl.BlockSpec((BR, H), lambda i: (i, 0)),
            pl.BlockSpec((BR, H), lambda i: (i + NPAD // BR, 0)),
            pl.BlockSpec((NS, BR), lambda i: (0, i)),
            pl.BlockSpec((BR, D_IN), lambda i: (i, 0)),
            pl.BlockSpec((H, D_IN), lambda i: (0, 0)),
            pl.BlockSpec((H, D_IN), lambda i: (0, 0)),
            pl.BlockSpec((1, D_IN), lambda i: (0, 0)),
        ],
        out_specs=pl.BlockSpec((BR, D_IN), lambda i: (i, 0)),
        out_shape=jax.ShapeDtypeStruct((N_NODES, D_IN), jnp.float32),
    )(agg, agg, deg_parts, features, W[:H], W[H:], b.reshape(1, D_IN))


def kernel(features, edges, W, b):
    src = edges[0].astype(jnp.int32)
    dst = edges[1].astype(jnp.int32)
    pad = EPAD - N_EDGES
    srcp = jnp.concatenate([src, jnp.zeros((pad,), jnp.int32)])
    dstp = jnp.concatenate([dst, jnp.full((pad,), N_NODES, jnp.int32)])
    idxpk = jnp.stack(
        [srcp.reshape(NS, NCHUNK, CHUNK), dstp.reshape(NS, NCHUNK, CHUNK)],
        axis=2,
    )
    zrows = jnp.zeros((ROWS_PER_TILE, H), jnp.float32)
    zhist = jnp.zeros((NPAD,), jnp.float32)
    agg, deg_parts = _sc_aggregate(features, idxpk, zrows, zhist)
    return _tc_finish(agg, deg_parts, features, W, b)


# speedup vs baseline: 2.6755x; 1.0078x over previous
import functools

import jax
import jax.numpy as jnp
from jax import lax
from jax.experimental import pallas as pl
from jax.experimental.pallas import tpu as pltpu
from jax.experimental.pallas import tpu_sc as plsc

N_NODES = 10000
N_EDGES = 160000
D_IN = 256
H = 128
NC = 2
NS = 16
L = 16
CHUNK = 128
DEPTH = 2
EDGES_PER_TILE = 10240
EPAD = NS * EDGES_PER_TILE
NCHUNK = EDGES_PER_TILE // CHUNK
ROWS_PER_TILE = 640
NPAD = NS * ROWS_PER_TILE


def _sc_aggregate(features, idxpk, zrows, zhist):
    mesh = plsc.VectorSubcoreMesh(
        core_axis_name="c", subcore_axis_name="s", num_cores=NC, num_subcores=NS
    )

    @functools.partial(
        pl.kernel,
        out_type=[
            jax.ShapeDtypeStruct((NC * NPAD, H), jnp.float32),
            jax.ShapeDtypeStruct((NS, NPAD), jnp.float32),
        ],
        mesh=mesh,
        compiler_params=pltpu.CompilerParams(needs_layout_passes=False),
        scratch_types=(
            [pltpu.VMEM((2, CHUNK), jnp.int32) for _ in range(DEPTH)]
            + [pltpu.VMEM((CHUNK, H), jnp.float32) for _ in range(DEPTH)]
            + [
                pltpu.VMEM((NPAD,), jnp.float32),
                pltpu.VMEM_SHARED((NPAD, H), jnp.float32),
            ]
            + [pltpu.SemaphoreType.DMA for _ in range(2 * DEPTH)]
        ),
    )
    def body(feat_hbm, idx_hbm, z_hbm, zh_hbm, agg_hbm, deg_hbm, *refs):
        ibufs = refs[0:DEPTH]
        rowss = refs[DEPTH:2 * DEPTH]
        hist = refs[2 * DEPTH]
        acc = refs[2 * DEPTH + 1]
        isems = refs[2 * DEPTH + 2:2 * DEPTH + 2 + DEPTH]
        gsems = refs[2 * DEPTH + 2 + DEPTH:]
        c = lax.axis_index("c")
        s = lax.axis_index("s")
        colb = pl.multiple_of(c * H, H)
        pltpu.sync_copy(z_hbm, acc.at[pl.ds(s * ROWS_PER_TILE, ROWS_PER_TILE)])
        pltpu.sync_copy(zh_hbm, hist)
        plsc.subcore_barrier()

        def idx_load(k, slot):
            return pltpu.async_copy(idx_hbm.at[s, k], ibufs[slot], isems[slot])

        def idx_wait(k, slot):
            pltpu.make_async_copy(idx_hbm.at[s, k], ibufs[slot],
                                  isems[slot]).wait()

        def gather(slot):
            return pltpu.async_copy(
                feat_hbm.at[ibufs[slot].at[0], pl.ds(colb, H)], rowss[slot],
                gsems[slot])

        def gather_wait(slot):
            pltpu.make_async_copy(
                feat_hbm.at[ibufs[slot].at[0], pl.ds(colb, H)], rowss[slot],
                gsems[slot]).wait()

        for i in range(DEPTH):
            idx_load(i, i)
        for i in range(DEPTH - 1):
            idx_wait(i, i)
            gather(i)

        def group(m, _):
            for b in range(DEPTH):
                k = m * DEPTH + b
                lead = (b + DEPTH - 1) % DEPTH
                @pl.when(k + DEPTH - 1 < NCHUNK)
                def _():
                    idx_wait(k + DEPTH - 1, lead)
                    gather(lead)
                for j in range(CHUNK // L):
                    d = ibufs[b][1, pl.ds(j * L, L)]
                    cnt, last = plsc.scan_count(d)
                    plsc.addupdate_scatter(hist, [d], cnt.astype(jnp.float32),
                                           mask=last)
                gather_wait(b)
                pltpu.sync_copy(rowss[b], acc.at[ibufs[b].at[1]], add=True)
                @pl.when(k + DEPTH < NCHUNK)
                def _():
                    idx_load(k + DEPTH, b)
            return 0

        lax.fori_loop(0, NCHUNK // DEPTH, group, 0)
        plsc.subcore_barrier()
        pltpu.sync_copy(
            acc.at[pl.ds(s * ROWS_PER_TILE, ROWS_PER_TILE)],
            agg_hbm.at[pl.ds(c * NPAD + s * ROWS_PER_TILE, ROWS_PER_TILE)],
        )
        @pl.when(c == 0)
        def _():
            pltpu.sync_copy(hist, deg_hbm.at[s])

    return body(features, idxpk, zrows, zhist)


BR = ROWS_PER_TILE


def _tc_body(aggA, aggB, degp, feat, w0, w1, b, out):
    deg = jnp.sum(degp[...], axis=0)[:, None]
    scale = 1.0 / jnp.maximum(deg, 1.0)
    h = jnp.dot(aggA[...] * scale, w0[...], preferred_element_type=jnp.float32)
    h = h + jnp.dot(aggB[...] * scale, w1[...], preferred_element_type=jnp.float32)
    out[...] = jnp.maximum(h + b[...] + feat[...], 0.0)


def _tc_finish(agg, deg_parts, features, W, b):
    grid = (NPAD // BR,)
    return pl.pallas_call(
        _tc_body,
        grid=grid,
        in_specs=[
            pl.BlockSpec((BR, H), lambda i: (i, 0)),
            pl.BlockSpec((BR, H), lambda i: (i + NPAD // BR, 0)),
            pl.BlockSpec((NS, BR), lambda i: (0, i)),
            pl.BlockSpec((BR, D_IN), lambda i: (i, 0)),
            pl.BlockSpec((H, D_IN), lambda i: (0, 0)),
            pl.BlockSpec((H, D_IN), lambda i: (0, 0)),
            pl.BlockSpec((1, D_IN), lambda i: (0, 0)),
        ],
        out_specs=pl.BlockSpec((BR, D_IN), lambda i: (i, 0)),
        out_shape=jax.ShapeDtypeStruct((N_NODES, D_IN), jnp.float32),
    )(agg, agg, deg_parts, features, W[:H], W[H:], b.reshape(1, D_IN))


def kernel(features, edges, W, b):
    src = edges[0].astype(jnp.int32)
    dst = edges[1].astype(jnp.int32)
    pad = EPAD - N_EDGES
    srcp = jnp.concatenate([src, jnp.zeros((pad,), jnp.int32)])
    dstp = jnp.concatenate([dst, jnp.full((pad,), N_NODES, jnp.int32)])
    idxpk = jnp.stack(
        [srcp.reshape(NS, NCHUNK, CHUNK), dstp.reshape(NS, NCHUNK, CHUNK)],
        axis=2,
    )
    zrows = jnp.zeros((ROWS_PER_TILE, H), jnp.float32)
    zhist = jnp.zeros((NPAD,), jnp.float32)
    agg, deg_parts = _sc_aggregate(features, idxpk, zrows, zhist)
    return _tc_finish(agg, deg_parts, features, W, b)
